# Initial kernel scaffold; baseline (speedup 1.0000x reference)
import jax
import jax.numpy as jnp
from jax.experimental import pallas as pl


def kernel(x, adj, W1, b1, W2, b2):
    raise NotImplementedError("write your pallas kernel here")


# speedup vs baseline: 2.3709x; 2.3709x over previous
import functools

import jax
import jax.numpy as jnp
from jax.experimental import pallas as pl
from jax.experimental.pallas import tpu as pltpu

_BM = 512


def _sym_deg_kernel(adj_ij, adj_ji, a_out, deg_out, *, bm, n):
    i = pl.program_id(0)
    j = pl.program_id(1)
    a = adj_ij[...]
    at = adj_ji[...].T
    rid = jax.lax.broadcasted_iota(jnp.int32, (bm, bm), 0) + i * bm
    cid = jax.lax.broadcasted_iota(jnp.int32, (bm, bm), 1) + j * bm
    valid = (rid < n) & (cid < n)
    sym = (a != 0.0) | (at != 0.0)
    ab = jnp.where(valid & sym, 1.0, 0.0)
    a_out[...] = ab.astype(jnp.bfloat16)
    rs = jnp.sum(ab, axis=1, keepdims=True)

    @pl.when(j == 0)
    def _():
        deg_out[...] = rs

    @pl.when(j > 0)
    def _():
        deg_out[...] += rs


def _z1_kernel(x_ref, w1_ref, dinv_ref, zhi_ref, zlo_ref, *, bm, n):
    i = pl.program_id(0)
    t = jnp.dot(x_ref[...], w1_ref[...], preferred_element_type=jnp.float32)
    t = t * dinv_ref[...]
    rid = jax.lax.broadcasted_iota(jnp.int32, (bm, 1), 0) + i * bm
    t = jnp.where(rid < n, t, 0.0)
    hi = t.astype(jnp.bfloat16)
    zhi_ref[...] = hi
    zlo_ref[...] = (t - hi.astype(jnp.float32)).astype(jnp.bfloat16)


def _layer1_kernel(a_ref, zhi_ref, zlo_ref, dinv_ref, b1_ref, w2_ref,
                   z2hi_ref, z2lo_ref, acc_ref, *, nj):
    j = pl.program_id(1)

    @pl.when(j == 0)
    def _():
        acc_ref[...] = jnp.zeros_like(acc_ref)

    a = a_ref[...]
    acc_ref[...] += (
        jnp.dot(a, zhi_ref[...], preferred_element_type=jnp.float32)
        + jnp.dot(a, zlo_ref[...], preferred_element_type=jnp.float32))

    @pl.when(j == nj - 1)
    def _():
        dinv = dinv_ref[...]
        h = acc_ref[...] * dinv + b1_ref[...]
        h = jnp.maximum(h, 0.0)
        z2 = jnp.dot(h, w2_ref[...], preferred_element_type=jnp.float32) * dinv
        hi = z2.astype(jnp.bfloat16)
        z2hi_ref[...] = hi
        z2lo_ref[...] = (z2 - hi.astype(jnp.float32)).astype(jnp.bfloat16)


def _layer2_kernel(a_ref, zhi_ref, zlo_ref, dinv_ref, b2_ref, out_ref,
                   acc_ref, *, nj):
    j = pl.program_id(1)

    @pl.when(j == 0)
    def _():
        acc_ref[...] = jnp.zeros_like(acc_ref)

    a = a_ref[...]
    acc_ref[...] += (
        jnp.dot(a, zhi_ref[...], preferred_element_type=jnp.float32)
        + jnp.dot(a, zlo_ref[...], preferred_element_type=jnp.float32))

    @pl.when(j == nj - 1)
    def _():
        y = acc_ref[...] * dinv_ref[...] + b2_ref[...]
        m = jnp.max(y, axis=1, keepdims=True)
        lse = m + jnp.log(jnp.sum(jnp.exp(y - m), axis=1, keepdims=True))
        out_ref[...] = y - lse


def kernel(x, adj, W1, b1, W2, b2):
    n = adj.shape[0]
    f_in = x.shape[1]
    f_hid = W1.shape[1]
    f_out = W2.shape[1]
    bm = _BM
    ni = pl.cdiv(n, bm)
    npad = ni * bm

    a_mat, deg = pl.pallas_call(
        functools.partial(_sym_deg_kernel, bm=bm, n=n),
        grid=(ni, ni),
        in_specs=[
            pl.BlockSpec((bm, bm), lambda i, j: (i, j)),
            pl.BlockSpec((bm, bm), lambda i, j: (j, i)),
        ],
        out_specs=[
            pl.BlockSpec((bm, bm), lambda i, j: (i, j)),
            pl.BlockSpec((bm, 1), lambda i, j: (i, 0)),
        ],
        out_shape=[
            jax.ShapeDtypeStruct((npad, npad), jnp.bfloat16),
            jax.ShapeDtypeStruct((npad, 1), jnp.float32),
        ],
        compiler_params=pltpu.CompilerParams(
            dimension_semantics=("parallel", "arbitrary")),
    )(adj, adj)

    dinv = jnp.where(deg > 0.0, jax.lax.rsqrt(jnp.maximum(deg, 1e-12)), 0.0)

    z1hi, z1lo = pl.pallas_call(
        functools.partial(_z1_kernel, bm=bm, n=n),
        grid=(ni,),
        in_specs=[
            pl.BlockSpec((bm, f_in), lambda i: (i, 0)),
            pl.BlockSpec((f_in, f_hid), lambda i: (0, 0)),
            pl.BlockSpec((bm, 1), lambda i: (i, 0)),
        ],
        out_specs=[
            pl.BlockSpec((bm, f_hid), lambda i: (i, 0)),
            pl.BlockSpec((bm, f_hid), lambda i: (i, 0)),
        ],
        out_shape=[
            jax.ShapeDtypeStruct((npad, f_hid), jnp.bfloat16),
            jax.ShapeDtypeStruct((npad, f_hid), jnp.bfloat16),
        ],
        compiler_params=pltpu.CompilerParams(
            dimension_semantics=("parallel",)),
    )(x, W1, dinv)

    z2hi, z2lo = pl.pallas_call(
        functools.partial(_layer1_kernel, nj=ni),
        grid=(ni, ni),
        in_specs=[
            pl.BlockSpec((bm, bm), lambda i, j: (i, j)),
            pl.BlockSpec((bm, f_hid), lambda i, j: (j, 0)),
            pl.BlockSpec((bm, f_hid), lambda i, j: (j, 0)),
            pl.BlockSpec((bm, 1), lambda i, j: (i, 0)),
            pl.BlockSpec((1, f_hid), lambda i, j: (0, 0)),
            pl.BlockSpec((f_hid, f_out), lambda i, j: (0, 0)),
        ],
        out_specs=[
            pl.BlockSpec((bm, f_out), lambda i, j: (i, 0)),
            pl.BlockSpec((bm, f_out), lambda i, j: (i, 0)),
        ],
        out_shape=[
            jax.ShapeDtypeStruct((npad, f_out), jnp.bfloat16),
            jax.ShapeDtypeStruct((npad, f_out), jnp.bfloat16),
        ],
        scratch_shapes=[pltpu.VMEM((bm, f_hid), jnp.float32)],
        compiler_params=pltpu.CompilerParams(
            dimension_semantics=("parallel", "arbitrary")),
    )(a_mat, z1hi, z1lo, dinv, b1.reshape(1, f_hid), W2)

    out = pl.pallas_call(
        functools.partial(_layer2_kernel, nj=ni),
        grid=(ni, ni),
        in_specs=[
            pl.BlockSpec((bm, bm), lambda i, j: (i, j)),
            pl.BlockSpec((bm, f_out), lambda i, j: (j, 0)),
            pl.BlockSpec((bm, f_out), lambda i, j: (j, 0)),
            pl.BlockSpec((bm, 1), lambda i, j: (i, 0)),
            pl.BlockSpec((1, f_out), lambda i, j: (0, 0)),
        ],
        out_specs=pl.BlockSpec((bm, f_out), lambda i, j: (i, 0)),
        out_shape=jax.ShapeDtypeStruct((npad, f_out), jnp.float32),
        scratch_shapes=[pltpu.VMEM((bm, f_out), jnp.float32)],
        compiler_params=pltpu.CompilerParams(
            dimension_semantics=("parallel", "arbitrary")),
    )(a_mat, z2hi, z2lo, dinv, b2.reshape(1, f_out))

    return out[:n]


# speedup vs baseline: 3.4752x; 1.4658x over previous
import functools

import jax
import jax.numpy as jnp
from jax.experimental import pallas as pl
from jax.experimental.pallas import tpu as pltpu

_BM = 512

_T_DIMS = (((0,), (0,)), ((), ()))


def _sym_deg_kernel(adj_ij, adj_ji, a_out, dinv_out, deg_acc, *, bm, n, ni):
    i = pl.program_id(0)
    j = pl.program_id(1)

    @pl.when((i == 0) & (j == 0))
    def _():
        deg_acc[...] = jnp.zeros_like(deg_acc)

    @pl.when(j >= i)
    def _():
        a = adj_ij[...]
        at = adj_ji[...].T
        m = jnp.maximum(a, at)

        is_edge = (i == ni - 1) | (j == ni - 1)

        @pl.when(is_edge)
        def _():
            rid = jax.lax.broadcasted_iota(jnp.int32, (bm, 1), 0)
            cid = jax.lax.broadcasted_iota(jnp.int32, (1, bm), 1)
            valid = (rid < n - i * bm) & (cid < n - j * bm)
            a_out[...] = jnp.where(valid, m, 0.0).astype(jnp.bfloat16)

        @pl.when(~is_edge)
        def _():
            a_out[...] = m.astype(jnp.bfloat16)

        ab = a_out[...]
        ones = jnp.ones((bm, 1), dtype=jnp.bfloat16)
        rs = jnp.dot(ab, ones, preferred_element_type=jnp.float32)
        deg_acc[pl.ds(i * bm, bm), :] += rs

        @pl.when(j > i)
        def _():
            cs = jax.lax.dot_general(ab, ones, _T_DIMS,
                                     preferred_element_type=jnp.float32)
            deg_acc[pl.ds(j * bm, bm), :] += cs

    @pl.when((i == ni - 1) & (j == ni - 1))
    def _():
        deg = deg_acc[...]
        dinv_out[...] = jnp.where(
            deg > 0.0, jax.lax.rsqrt(jnp.maximum(deg, 1e-12)), 0.0)


def _z1_kernel(x_ref, w1_ref, dinv_ref, zhi_ref, zlo_ref, *, bm, n):
    i = pl.program_id(0)
    t = jnp.dot(x_ref[...], w1_ref[...], preferred_element_type=jnp.float32)
    t = t * dinv_ref[...]
    rid = jax.lax.broadcasted_iota(jnp.int32, (bm, 1), 0) + i * bm
    t = jnp.where(rid < n, t, 0.0)
    hi = t.astype(jnp.bfloat16)
    zhi_ref[...] = hi
    zlo_ref[...] = (t - hi.astype(jnp.float32)).astype(jnp.bfloat16)


def _acc_sym(a_ref, zhi_j, zlo_j, zhi_i, zlo_i, acc_ref, i, j, bm):
    a = a_ref[...]
    u = (jnp.dot(a, zhi_j[...], preferred_element_type=jnp.float32)
         + jnp.dot(a, zlo_j[...], preferred_element_type=jnp.float32))
    acc_ref[pl.ds(i * bm, bm), :] += u

    @pl.when(j > i)
    def _():
        v = (jax.lax.dot_general(a, zhi_i[...], _T_DIMS,
                                 preferred_element_type=jnp.float32)
             + jax.lax.dot_general(a, zlo_i[...], _T_DIMS,
                                   preferred_element_type=jnp.float32))
        acc_ref[pl.ds(j * bm, bm), :] += v


def _layer1_kernel(a_ref, zhi_j, zlo_j, zhi_i, zlo_i, dinv_ref, b1_ref,
                   w2_ref, z2hi_ref, z2lo_ref, acc_ref, *, bm, ni):
    i = pl.program_id(0)
    j = pl.program_id(1)

    @pl.when((i == 0) & (j == 0))
    def _():
        acc_ref[...] = jnp.zeros_like(acc_ref)

    @pl.when(j >= i)
    def _():
        _acc_sym(a_ref, zhi_j, zlo_j, zhi_i, zlo_i, acc_ref, i, j, bm)

    @pl.when((i == ni - 1) & (j == ni - 1))
    def _():
        dinv = dinv_ref[...]
        h = acc_ref[...] * dinv + b1_ref[...]
        h = jnp.maximum(h, 0.0)
        z2 = jnp.dot(h, w2_ref[...], preferred_element_type=jnp.float32) * dinv
        hi = z2.astype(jnp.bfloat16)
        z2hi_ref[...] = hi
        z2lo_ref[...] = (z2 - hi.astype(jnp.float32)).astype(jnp.bfloat16)


def _layer2_kernel(a_ref, zhi_j, zlo_j, zhi_i, zlo_i, dinv_ref, b2_ref,
                   out_ref, acc_ref, *, bm, ni):
    i = pl.program_id(0)
    j = pl.program_id(1)

    @pl.when((i == 0) & (j == 0))
    def _():
        acc_ref[...] = jnp.zeros_like(acc_ref)

    @pl.when(j >= i)
    def _():
        _acc_sym(a_ref, zhi_j, zlo_j, zhi_i, zlo_i, acc_ref, i, j, bm)

    @pl.when((i == ni - 1) & (j == ni - 1))
    def _():
        y = acc_ref[...] * dinv_ref[...] + b2_ref[...]
        m = jnp.max(y, axis=1, keepdims=True)
        lse = m + jnp.log(jnp.sum(jnp.exp(y - m), axis=1, keepdims=True))
        out_ref[...] = y - lse


def kernel(x, adj, W1, b1, W2, b2):
    n = adj.shape[0]
    f_in = x.shape[1]
    f_hid = W1.shape[1]
    f_out = W2.shape[1]
    bm = _BM
    ni = pl.cdiv(n, bm)
    npad = ni * bm

    a_mat, dinv = pl.pallas_call(
        functools.partial(_sym_deg_kernel, bm=bm, n=n, ni=ni),
        grid=(ni, ni),
        in_specs=[
            pl.BlockSpec((bm, bm), lambda i, j: (i, jnp.maximum(i, j))),
            pl.BlockSpec((bm, bm), lambda i, j: (jnp.maximum(i, j), i)),
        ],
        out_specs=[
            pl.BlockSpec((bm, bm), lambda i, j: (i, jnp.maximum(i, j))),
            pl.BlockSpec((npad, 1), lambda i, j: (0, 0)),
        ],
        out_shape=[
            jax.ShapeDtypeStruct((npad, npad), jnp.bfloat16),
            jax.ShapeDtypeStruct((npad, 1), jnp.float32),
        ],
        scratch_shapes=[pltpu.VMEM((npad, 1), jnp.float32)],
        compiler_params=pltpu.CompilerParams(
            dimension_semantics=("arbitrary", "arbitrary")),
    )(adj, adj)

    z1hi, z1lo = pl.pallas_call(
        functools.partial(_z1_kernel, bm=bm, n=n),
        grid=(ni,),
        in_specs=[
            pl.BlockSpec((bm, f_in), lambda i: (i, 0)),
            pl.BlockSpec((f_in, f_hid), lambda i: (0, 0)),
            pl.BlockSpec((bm, 1), lambda i: (i, 0)),
        ],
        out_specs=[
            pl.BlockSpec((bm, f_hid), lambda i: (i, 0)),
            pl.BlockSpec((bm, f_hid), lambda i: (i, 0)),
        ],
        out_shape=[
            jax.ShapeDtypeStruct((npad, f_hid), jnp.bfloat16),
            jax.ShapeDtypeStruct((npad, f_hid), jnp.bfloat16),
        ],
        compiler_params=pltpu.CompilerParams(
            dimension_semantics=("parallel",)),
    )(x, W1, dinv)

    def _sym_specs(fdim):
        return [
            pl.BlockSpec((bm, bm), lambda i, j: (i, jnp.maximum(i, j))),
            pl.BlockSpec((bm, fdim), lambda i, j: (jnp.maximum(i, j), 0)),
            pl.BlockSpec((bm, fdim), lambda i, j: (jnp.maximum(i, j), 0)),
            pl.BlockSpec((bm, fdim), lambda i, j: (i, 0)),
            pl.BlockSpec((bm, fdim), lambda i, j: (i, 0)),
            pl.BlockSpec((npad, 1), lambda i, j: (0, 0)),
        ]

    z2hi, z2lo = pl.pallas_call(
        functools.partial(_layer1_kernel, bm=bm, ni=ni),
        grid=(ni, ni),
        in_specs=_sym_specs(f_hid) + [
            pl.BlockSpec((1, f_hid), lambda i, j: (0, 0)),
            pl.BlockSpec((f_hid, f_out), lambda i, j: (0, 0)),
        ],
        out_specs=[
            pl.BlockSpec((npad, f_out), lambda i, j: (0, 0)),
            pl.BlockSpec((npad, f_out), lambda i, j: (0, 0)),
        ],
        out_shape=[
            jax.ShapeDtypeStruct((npad, f_out), jnp.bfloat16),
            jax.ShapeDtypeStruct((npad, f_out), jnp.bfloat16),
        ],
        scratch_shapes=[pltpu.VMEM((npad, f_hid), jnp.float32)],
        compiler_params=pltpu.CompilerParams(
            dimension_semantics=("arbitrary", "arbitrary")),
    )(a_mat, z1hi, z1lo, z1hi, z1lo, dinv, b1.reshape(1, f_hid), W2)

    out = pl.pallas_call(
        functools.partial(_layer2_kernel, bm=bm, ni=ni),
        grid=(ni, ni),
        in_specs=_sym_specs(f_out) + [
            pl.BlockSpec((1, f_out), lambda i, j: (0, 0)),
        ],
        out_specs=pl.BlockSpec((npad, f_out), lambda i, j: (0, 0)),
        out_shape=jax.ShapeDtypeStruct((npad, f_out), jnp.float32),
        scratch_shapes=[pltpu.VMEM((npad, f_out), jnp.float32)],
        compiler_params=pltpu.CompilerParams(
            dimension_semantics=("arbitrary", "arbitrary")),
    )(a_mat, z2hi, z2lo, z2hi, z2lo, dinv, b2.reshape(1, f_out))

    return out[:n]


# speedup vs baseline: 6.8771x; 1.9789x over previous
import functools

import jax
import jax.numpy as jnp
from jax.experimental import pallas as pl
from jax.experimental.pallas import tpu as pltpu

_BM = 1024

_T_DIMS = (((0,), (0,)), ((), ()))


def _sym_deg_kernel(adj_ij, adj_ji, a_out, dinv_out, deg_acc, *, bm, n, ni):
    i = pl.program_id(0)
    j = pl.program_id(1)

    @pl.when((i == 0) & (j == 0))
    def _():
        deg_acc[...] = jnp.zeros_like(deg_acc)

    def finish(mv):
        mb = mv.astype(jnp.bfloat16)
        a_out[...] = mb
        ones = jnp.ones((bm, 1), dtype=jnp.bfloat16)
        rs = jnp.dot(mb, ones, preferred_element_type=jnp.float32)
        deg_acc[pl.ds(i * bm, bm), :] += rs

        @pl.when(j > i)
        def _():
            cs = jax.lax.dot_general(mb, ones, _T_DIMS,
                                     preferred_element_type=jnp.float32)
            deg_acc[pl.ds(j * bm, bm), :] += cs

    @pl.when(j >= i)
    def _():
        a = adj_ij[...]
        at = adj_ji[...].T
        m = jnp.maximum(a, at)

        is_edge = (i == ni - 1) | (j == ni - 1)

        @pl.when(is_edge)
        def _():
            rid = jax.lax.broadcasted_iota(jnp.int32, (bm, 1), 0)
            cid = jax.lax.broadcasted_iota(jnp.int32, (1, bm), 1)
            valid = (rid < n - i * bm) & (cid < n - j * bm)
            finish(jnp.where(valid, m, 0.0))

        @pl.when(~is_edge)
        def _():
            finish(m)

    @pl.when((i == ni - 1) & (j == ni - 1))
    def _():
        deg = deg_acc[...]
        dinv_out[...] = jnp.where(
            deg > 0.0, jax.lax.rsqrt(jnp.maximum(deg, 1e-12)), 0.0)


def _split_hi_lo(t):
    hi = t.astype(jnp.bfloat16)
    lo = (t - hi.astype(jnp.float32)).astype(jnp.bfloat16)
    return jnp.concatenate([hi, lo], axis=1)


def _z1_kernel(x_ref, w1_ref, dinv_ref, z_ref, *, bm, n):
    i = pl.program_id(0)
    t = jnp.dot(x_ref[...], w1_ref[...], preferred_element_type=jnp.float32)
    t = t * dinv_ref[...]
    rid = jax.lax.broadcasted_iota(jnp.int32, (bm, 1), 0) + i * bm
    t = jnp.where(rid < n, t, 0.0)
    z_ref[...] = _split_hi_lo(t)


def _acc_sym(a_ref, z_j, z_i, acc_ref, i, j, bm):
    a = a_ref[...]
    u = jnp.dot(a, z_j[...], preferred_element_type=jnp.float32)
    acc_ref[pl.ds(i * bm, bm), :] += u

    @pl.when(j > i)
    def _():
        v = jax.lax.dot_general(a, z_i[...], _T_DIMS,
                                preferred_element_type=jnp.float32)
        acc_ref[pl.ds(j * bm, bm), :] += v


def _layer1_kernel(a_ref, z_j, z_i, dinv_ref, b1_ref, w2_ref,
                   z2_ref, acc_ref, *, bm, ni, f_hid):
    i = pl.program_id(0)
    j = pl.program_id(1)

    @pl.when((i == 0) & (j == 0))
    def _():
        acc_ref[...] = jnp.zeros_like(acc_ref)

    @pl.when(j >= i)
    def _():
        _acc_sym(a_ref, z_j, z_i, acc_ref, i, j, bm)

    @pl.when((i == ni - 1) & (j == ni - 1))
    def _():
        dinv = dinv_ref[...]
        acc = acc_ref[...]
        h = (acc[:, :f_hid] + acc[:, f_hid:]) * dinv + b1_ref[...]
        h = jnp.maximum(h, 0.0)
        z2 = jnp.dot(h, w2_ref[...], preferred_element_type=jnp.float32) * dinv
        z2_ref[...] = _split_hi_lo(z2)


def _layer2_kernel(a_ref, z_j, z_i, dinv_ref, b2_ref, out_ref, acc_ref,
                   *, bm, ni, f_out):
    i = pl.program_id(0)
    j = pl.program_id(1)

    @pl.when((i == 0) & (j == 0))
    def _():
        acc_ref[...] = jnp.zeros_like(acc_ref)

    @pl.when(j >= i)
    def _():
        _acc_sym(a_ref, z_j, z_i, acc_ref, i, j, bm)

    @pl.when((i == ni - 1) & (j == ni - 1))
    def _():
        acc = acc_ref[...]
        y = (acc[:, :f_out] + acc[:, f_out:]) * dinv_ref[...] + b2_ref[...]
        m = jnp.max(y, axis=1, keepdims=True)
        lse = m + jnp.log(jnp.sum(jnp.exp(y - m), axis=1, keepdims=True))
        out_ref[...] = y - lse


def kernel(x, adj, W1, b1, W2, b2):
    n = adj.shape[0]
    f_in = x.shape[1]
    f_hid = W1.shape[1]
    f_out = W2.shape[1]
    bm = _BM
    ni = pl.cdiv(n, bm)
    npad = ni * bm

    a_mat, dinv = pl.pallas_call(
        functools.partial(_sym_deg_kernel, bm=bm, n=n, ni=ni),
        grid=(ni, ni),
        in_specs=[
            pl.BlockSpec((bm, bm), lambda i, j: (i, jnp.maximum(i, j))),
            pl.BlockSpec((bm, bm), lambda i, j: (jnp.maximum(i, j), i)),
        ],
        out_specs=[
            pl.BlockSpec((bm, bm), lambda i, j: (i, jnp.maximum(i, j))),
            pl.BlockSpec((npad, 1), lambda i, j: (0, 0)),
        ],
        out_shape=[
            jax.ShapeDtypeStruct((npad, npad), jnp.bfloat16),
            jax.ShapeDtypeStruct((npad, 1), jnp.float32),
        ],
        scratch_shapes=[pltpu.VMEM((npad, 1), jnp.float32)],
        compiler_params=pltpu.CompilerParams(
            dimension_semantics=("arbitrary", "arbitrary")),
    )(adj, adj)

    z1 = pl.pallas_call(
        functools.partial(_z1_kernel, bm=bm, n=n),
        grid=(ni,),
        in_specs=[
            pl.BlockSpec((bm, f_in), lambda i: (i, 0)),
            pl.BlockSpec((f_in, f_hid), lambda i: (0, 0)),
            pl.BlockSpec((bm, 1), lambda i: (i, 0)),
        ],
        out_specs=pl.BlockSpec((bm, 2 * f_hid), lambda i: (i, 0)),
        out_shape=jax.ShapeDtypeStruct((npad, 2 * f_hid), jnp.bfloat16),
        compiler_params=pltpu.CompilerParams(
            dimension_semantics=("parallel",)),
    )(x, W1, dinv)

    def _sym_specs(fdim):
        return [
            pl.BlockSpec((bm, bm), lambda i, j: (i, jnp.maximum(i, j))),
            pl.BlockSpec((bm, 2 * fdim), lambda i, j: (jnp.maximum(i, j), 0)),
            pl.BlockSpec((bm, 2 * fdim), lambda i, j: (i, 0)),
            pl.BlockSpec((npad, 1), lambda i, j: (0, 0)),
        ]

    z2 = pl.pallas_call(
        functools.partial(_layer1_kernel, bm=bm, ni=ni, f_hid=f_hid),
        grid=(ni, ni),
        in_specs=_sym_specs(f_hid) + [
            pl.BlockSpec((1, f_hid), lambda i, j: (0, 0)),
            pl.BlockSpec((f_hid, f_out), lambda i, j: (0, 0)),
        ],
        out_specs=pl.BlockSpec((npad, 2 * f_out), lambda i, j: (0, 0)),
        out_shape=jax.ShapeDtypeStruct((npad, 2 * f_out), jnp.bfloat16),
        scratch_shapes=[pltpu.VMEM((npad, 2 * f_hid), jnp.float32)],
        compiler_params=pltpu.CompilerParams(
            dimension_semantics=("arbitrary", "arbitrary")),
    )(a_mat, z1, z1, dinv, b1.reshape(1, f_hid), W2)

    out = pl.pallas_call(
        functools.partial(_layer2_kernel, bm=bm, ni=ni, f_out=f_out),
        grid=(ni, ni),
        in_specs=_sym_specs(f_out) + [
            pl.BlockSpec((1, f_out), lambda i, j: (0, 0)),
        ],
        out_specs=pl.BlockSpec((npad, f_out), lambda i, j: (0, 0)),
        out_shape=jax.ShapeDtypeStruct((npad, f_out), jnp.float32),
        scratch_shapes=[pltpu.VMEM((npad, 2 * f_out), jnp.float32)],
        compiler_params=pltpu.CompilerParams(
            dimension_semantics=("arbitrary", "arbitrary")),
    )(a_mat, z2, z2, dinv, b2.reshape(1, f_out))

    return out[:n]


# speedup vs baseline: 7.7678x; 1.1295x over previous
import functools

import jax
import jax.numpy as jnp
from jax.experimental import pallas as pl
from jax.experimental.pallas import tpu as pltpu

_BM = 1024
_BL = 2048

_T_DIMS = (((0,), (0,)), ((), ()))


def _sym_deg_kernel(adj_ij, adj_ji, a_out, dinv_out, deg_r, deg_c,
                    *, bm, n, ni, r):
    i = pl.program_id(0)
    j = pl.program_id(1)

    @pl.when((i == 0) & (j == 0))
    def _():
        deg_r[...] = jnp.zeros_like(deg_r)
        deg_c[...] = jnp.zeros_like(deg_c)

    def finish(mv):
        a_out[...] = mv.astype(jnp.bfloat16)
        rs = jnp.sum(mv, axis=1, keepdims=True)
        deg_r[pl.ds(i * bm, bm), :] += rs

        @pl.when(j >= (i // r) * r + r)
        def _():
            cs = jnp.sum(mv, axis=0, keepdims=True)
            deg_c[:, pl.ds(j * bm, bm)] += cs

    @pl.when(j >= (i // r) * r)
    def _():
        a = adj_ij[...]
        at = adj_ji[...].T
        m = jnp.maximum(a, at)

        is_edge = ((i + 1) * bm > n) | ((j + 1) * bm > n)

        @pl.when(is_edge)
        def _():
            rid = jax.lax.broadcasted_iota(jnp.int32, (bm, 1), 0)
            cid = jax.lax.broadcasted_iota(jnp.int32, (1, bm), 1)
            valid = (rid < n - i * bm) & (cid < n - j * bm)
            finish(jnp.where(valid, m, 0.0))

        @pl.when(~is_edge)
        def _():
            finish(m)

    @pl.when((i == ni - 1) & (j == ni - 1))
    def _():
        deg = deg_r[...] + deg_c[...].T
        dinv_out[...] = jnp.where(
            deg > 0.0, jax.lax.rsqrt(jnp.maximum(deg, 1e-12)), 0.0)


def _split_hi_lo(t):
    hi = t.astype(jnp.bfloat16)
    lo = (t - hi.astype(jnp.float32)).astype(jnp.bfloat16)
    return jnp.concatenate([hi, lo], axis=1)


def _z1_kernel(x_ref, w1_ref, dinv_ref, z_ref, *, bm, n):
    i = pl.program_id(0)
    t = jnp.dot(x_ref[...], w1_ref[...], preferred_element_type=jnp.float32)
    t = t * dinv_ref[...]
    rid = jax.lax.broadcasted_iota(jnp.int32, (bm, 1), 0) + i * bm
    t = jnp.where(rid < n, t, 0.0)
    z_ref[...] = _split_hi_lo(t)


def _acc_sym(a_ref, z_j, z_i, acc_ref, i, j, bm):
    a = a_ref[...]
    u = jnp.dot(a, z_j[...], preferred_element_type=jnp.float32)
    acc_ref[pl.ds(i * bm, bm), :] += u

    @pl.when(j > i)
    def _():
        v = jax.lax.dot_general(a, z_i[...], _T_DIMS,
                                preferred_element_type=jnp.float32)
        acc_ref[pl.ds(j * bm, bm), :] += v


def _layer1_kernel(a_ref, z_j, z_i, dinv_ref, b1_ref, w2_ref,
                   z2_ref, acc_ref, *, bm, ni, f_hid):
    i = pl.program_id(0)
    j = pl.program_id(1)

    @pl.when((i == 0) & (j == 0))
    def _():
        acc_ref[...] = jnp.zeros_like(acc_ref)

    @pl.when(j >= i)
    def _():
        _acc_sym(a_ref, z_j, z_i, acc_ref, i, j, bm)

    @pl.when((i == ni - 1) & (j == ni - 1))
    def _():
        dinv = dinv_ref[...]
        acc = acc_ref[...]
        h = (acc[:, :f_hid] + acc[:, f_hid:]) * dinv + b1_ref[...]
        h = jnp.maximum(h, 0.0)
        z2 = jnp.dot(h, w2_ref[...], preferred_element_type=jnp.float32) * dinv
        z2_ref[...] = _split_hi_lo(z2)


def _layer2_kernel(a_ref, z_j, z_i, dinv_ref, b2_ref, out_ref, acc_ref,
                   *, bm, ni, f_out):
    i = pl.program_id(0)
    j = pl.program_id(1)

    @pl.when((i == 0) & (j == 0))
    def _():
        acc_ref[...] = jnp.zeros_like(acc_ref)

    @pl.when(j >= i)
    def _():
        _acc_sym(a_ref, z_j, z_i, acc_ref, i, j, bm)

    @pl.when((i == ni - 1) & (j == ni - 1))
    def _():
        acc = acc_ref[...]
        y = (acc[:, :f_out] + acc[:, f_out:]) * dinv_ref[...] + b2_ref[...]
        m = jnp.max(y, axis=1, keepdims=True)
        lse = m + jnp.log(jnp.sum(jnp.exp(y - m), axis=1, keepdims=True))
        out_ref[...] = y - lse


def kernel(x, adj, W1, b1, W2, b2):
    n = adj.shape[0]
    f_in = x.shape[1]
    f_hid = W1.shape[1]
    f_out = W2.shape[1]
    bm = _BM
    bl = _BL
    r = bl // bm
    npad = pl.cdiv(n, bl) * bl
    ni = npad // bm

    def _ja(i, j):
        return jnp.maximum(j, (i // r) * r)

    a_mat, dinv = pl.pallas_call(
        functools.partial(_sym_deg_kernel, bm=bm, n=n, ni=ni, r=r),
        grid=(ni, ni),
        in_specs=[
            pl.BlockSpec((bm, bm), lambda i, j: (i, _ja(i, j))),
            pl.BlockSpec((bm, bm), lambda i, j: (_ja(i, j), i)),
        ],
        out_specs=[
            pl.BlockSpec((bm, bm), lambda i, j: (i, _ja(i, j))),
            pl.BlockSpec((npad, 1), lambda i, j: (0, 0)),
        ],
        out_shape=[
            jax.ShapeDtypeStruct((npad, npad), jnp.bfloat16),
            jax.ShapeDtypeStruct((npad, 1), jnp.float32),
        ],
        scratch_shapes=[pltpu.VMEM((npad, 1), jnp.float32),
                        pltpu.VMEM((1, npad), jnp.float32)],
        compiler_params=pltpu.CompilerParams(
            dimension_semantics=("arbitrary", "arbitrary")),
    )(adj, adj)

    z1 = pl.pallas_call(
        functools.partial(_z1_kernel, bm=bm, n=n),
        grid=(ni,),
        in_specs=[
            pl.BlockSpec((bm, f_in), lambda i: (i, 0)),
            pl.BlockSpec((f_in, f_hid), lambda i: (0, 0)),
            pl.BlockSpec((bm, 1), lambda i: (i, 0)),
        ],
        out_specs=pl.BlockSpec((bm, 2 * f_hid), lambda i: (i, 0)),
        out_shape=jax.ShapeDtypeStruct((npad, 2 * f_hid), jnp.bfloat16),
        compiler_params=pltpu.CompilerParams(
            dimension_semantics=("parallel",)),
    )(x, W1, dinv)

    nl = npad // bl

    def _sym_specs(fdim):
        return [
            pl.BlockSpec((bl, bl), lambda i, j: (i, jnp.maximum(i, j))),
            pl.BlockSpec((bl, 2 * fdim), lambda i, j: (jnp.maximum(i, j), 0)),
            pl.BlockSpec((bl, 2 * fdim), lambda i, j: (i, 0)),
            pl.BlockSpec((npad, 1), lambda i, j: (0, 0)),
        ]

    z2 = pl.pallas_call(
        functools.partial(_layer1_kernel, bm=bl, ni=nl, f_hid=f_hid),
        grid=(nl, nl),
        in_specs=_sym_specs(f_hid) + [
            pl.BlockSpec((1, f_hid), lambda i, j: (0, 0)),
            pl.BlockSpec((f_hid, f_out), lambda i, j: (0, 0)),
        ],
        out_specs=pl.BlockSpec((npad, 2 * f_out), lambda i, j: (0, 0)),
        out_shape=jax.ShapeDtypeStruct((npad, 2 * f_out), jnp.bfloat16),
        scratch_shapes=[pltpu.VMEM((npad, 2 * f_hid), jnp.float32)],
        compiler_params=pltpu.CompilerParams(
            dimension_semantics=("arbitrary", "arbitrary")),
    )(a_mat, z1, z1, dinv, b1.reshape(1, f_hid), W2)

    out = pl.pallas_call(
        functools.partial(_layer2_kernel, bm=bl, ni=nl, f_out=f_out),
        grid=(nl, nl),
        in_specs=_sym_specs(f_out) + [
            pl.BlockSpec((1, f_out), lambda i, j: (0, 0)),
        ],
        out_specs=pl.BlockSpec((npad, f_out), lambda i, j: (0, 0)),
        out_shape=jax.ShapeDtypeStruct((npad, f_out), jnp.float32),
        scratch_shapes=[pltpu.VMEM((npad, 2 * f_out), jnp.float32)],
        compiler_params=pltpu.CompilerParams(
            dimension_semantics=("arbitrary", "arbitrary")),
    )(a_mat, z2, z2, dinv, b2.reshape(1, f_out))

    return out[:n]
